# baseline (speedup 1.0000x reference)
import functools
import math

import jax
import jax.numpy as jnp
from jax import lax
from jax.experimental import pallas as pl
from jax.experimental.pallas import tpu as pltpu
from jax.experimental.pallas import tpu_sc as plsc

EMBED_DIM = 64
SCALE = math.sqrt(EMBED_DIM)

NUM_CORES = 2
NUM_SUBCORES = 16
NUM_WORKERS = NUM_CORES * NUM_SUBCORES

CHUNK = 128
NBUF = 5
LANES = 16


def _build(total_rows: int):
    rows_per_worker = total_rows // NUM_WORKERS
    chunks_per_worker = rows_per_worker // CHUNK
    rounds = chunks_per_worker // NBUF
    assert rounds * NBUF == chunks_per_worker

    mesh = plsc.VectorSubcoreMesh(core_axis_name="c", subcore_axis_name="s")

    @functools.partial(
        pl.kernel,
        out_type=jax.ShapeDtypeStruct((total_rows, EMBED_DIM), jnp.float32),
        mesh=mesh,
        scratch_types=(
            [pltpu.VMEM((chunks_per_worker, CHUNK), jnp.int32)]
            + [pltpu.VMEM((CHUNK, EMBED_DIM), jnp.float32)] * NBUF
            + [pltpu.SemaphoreType.DMA] * (2 * NBUF)
        ),
        compiler_params=pltpu.CompilerParams(needs_layout_passes=False,
                                             disable_bounds_checks=True),
    )
    def emb(tokens_hbm, table_hbm, out_hbm, *scratch):
        ids_all = scratch[0]
        row_bufs = scratch[1:1 + NBUF]
        sem_in = scratch[1 + NBUF:1 + 2 * NBUF]
        sem_out = scratch[1 + 2 * NBUF:]

        wid = lax.axis_index("s") * NUM_CORES + lax.axis_index("c")
        base_row = wid * rows_per_worker

        pltpu.sync_copy(
            tokens_hbm.at[pl.ds(wid * chunks_per_worker, chunks_per_worker)],
            ids_all)

        def round_body(p, carry):
            c0 = p * NBUF

            for b in range(NBUF):
                rb = row_bufs[b]
                for grp in range(CHUNK // LANES):
                    vec = ids_all[c0 + b, pl.ds(grp * LANES, LANES)]
                    for l in range(LANES):
                        pltpu.async_copy(
                            table_hbm.at[vec[l]],
                            rb.at[grp * LANES + l],
                            sem_in[b])

            writebacks = []
            for b in range(NBUF):
                rb = row_bufs[b]
                pltpu.make_async_copy(
                    table_hbm.at[pl.ds(0, CHUNK)], rb, sem_in[b]).wait()

                @plsc.parallel_loop(0, CHUNK, 1, unroll=4)
                def scale_row(i):
                    for j in range(EMBED_DIM // LANES):
                        sl = (i, pl.ds(j * LANES, LANES))
                        rb[sl] = rb[sl] * SCALE

                writebacks.append(pltpu.async_copy(
                    rb,
                    out_hbm.at[pl.ds(base_row + (c0 + b) * CHUNK, CHUNK)],
                    sem_out[b]))
            for wb in writebacks:
                wb.wait()
            return carry

        lax.fori_loop(0, rounds, round_body, 0)

    return emb


def kernel(tokens, table):
    b, s = tokens.shape
    total_rows = b * s
    tokens2d = tokens.reshape(total_rows // CHUNK, CHUNK)
    out = _build(total_rows)(tokens2d, table)
    return out.reshape(b, s, EMBED_DIM)


# speedup vs baseline: 1.0059x; 1.0059x over previous
import functools
import math

import jax
import jax.numpy as jnp
from jax import lax
from jax.experimental import pallas as pl
from jax.experimental.pallas import tpu as pltpu
from jax.experimental.pallas import tpu_sc as plsc

EMBED_DIM = 64
SCALE = math.sqrt(EMBED_DIM)

NUM_CORES = 2
NUM_SUBCORES = 16
NUM_WORKERS = NUM_CORES * NUM_SUBCORES

CHUNK = 128
NBUF = 4
LANES = 16


def _build(total_rows: int):
    rows_per_worker = total_rows // NUM_WORKERS
    chunks_per_worker = rows_per_worker // CHUNK
    rounds = chunks_per_worker // NBUF
    assert rounds * NBUF == chunks_per_worker

    mesh = plsc.VectorSubcoreMesh(core_axis_name="c", subcore_axis_name="s")

    @functools.partial(
        pl.kernel,
        out_type=jax.ShapeDtypeStruct((total_rows, EMBED_DIM), jnp.float32),
        mesh=mesh,
        scratch_types=(
            [pltpu.VMEM((chunks_per_worker, CHUNK), jnp.int32)]
            + [pltpu.VMEM((CHUNK, EMBED_DIM), jnp.float32)] * NBUF
            + [pltpu.SemaphoreType.DMA] * (2 * NBUF)
        ),
        compiler_params=pltpu.CompilerParams(needs_layout_passes=False),
    )
    def emb(tokens_hbm, table_hbm, out_hbm, *scratch):
        ids_all = scratch[0]
        row_bufs = scratch[1:1 + NBUF]
        sem_in = scratch[1 + NBUF:1 + 2 * NBUF]
        sem_out = scratch[1 + 2 * NBUF:]

        wid = lax.axis_index("s") * NUM_CORES + lax.axis_index("c")
        base_row = wid * rows_per_worker

        pltpu.sync_copy(
            tokens_hbm.at[pl.ds(wid * chunks_per_worker, chunks_per_worker)],
            ids_all)

        def round_body(p, carry):
            c0 = p * NBUF

            for b in range(NBUF):
                rb = row_bufs[b]
                for grp in range(CHUNK // LANES):
                    vec = ids_all[c0 + b, pl.ds(grp * LANES, LANES)]
                    for l in range(LANES):
                        pltpu.async_copy(
                            table_hbm.at[vec[l]],
                            rb.at[grp * LANES + l],
                            sem_in[b])

            writebacks = []
            for b in range(NBUF):
                rb = row_bufs[b]
                pltpu.make_async_copy(
                    table_hbm.at[pl.ds(0, CHUNK)], rb, sem_in[b]).wait()

                @plsc.parallel_loop(0, CHUNK, 1, unroll=4)
                def scale_row(i):
                    for j in range(EMBED_DIM // LANES):
                        sl = (i, pl.ds(j * LANES, LANES))
                        rb[sl] = rb[sl] * SCALE

                writebacks.append(pltpu.async_copy(
                    rb,
                    out_hbm.at[pl.ds(base_row + (c0 + b) * CHUNK, CHUNK)],
                    sem_out[b]))
            for wb in writebacks:
                wb.wait()
            return carry

        lax.fori_loop(0, rounds, round_body, 0)

    return emb


def kernel(tokens, table):
    b, s = tokens.shape
    total_rows = b * s
    tokens2d = tokens.reshape(total_rows // CHUNK, CHUNK)
    out = _build(total_rows)(tokens2d, table)
    return out.reshape(b, s, EMBED_DIM)
